# Initial kernel scaffold; baseline (speedup 1.0000x reference)
import jax
import jax.numpy as jnp
from jax.experimental import pallas as pl


def kernel(z, W1, b1, g1, be1, W2, b2, g2, be2, W3, b3, edge_index):
    raise NotImplementedError("write your pallas kernel here")


# speedup vs baseline: 7.4094x; 7.4094x over previous
import functools

import jax
import jax.numpy as jnp
from jax import lax
from jax.experimental import pallas as pl
from jax.experimental.pallas import tpu as pltpu
from jax.experimental.pallas import tpu_sc as plsc

N = 10000
D = 128
NP = 10240
NB = NP // 128
NC, NS = 2, 16
NW = NC * NS
B = 128
PAD_ROW = N + 16

_mesh = plsc.VectorSubcoreMesh(core_axis_name="c", subcore_axis_name="s")


def _deg_body(dst_hbm, zeros_hbm, out_hbm, dst_v, hist_v, idx80_v, bounce_v,
              shared_deg):
    c = lax.axis_index("c")
    s = lax.axis_index("s")
    wid = s * NC + c
    epw = dst_hbm.shape[0] // NW
    pltpu.sync_copy(dst_hbm.at[pl.ds(wid * epw, epw)], dst_v)
    pltpu.sync_copy(zeros_hbm.at[pl.ds(0, NB)], hist_v)

    ones16 = jnp.ones((16,), jnp.float32)

    def hbody(i, carry):
        idx16 = dst_v[pl.ds(i * 16, 16)]
        r = lax.shift_right_logical(idx16, 7)
        col = lax.bitwise_and(idx16, 127)
        plsc.addupdate_scatter(hist_v, [r, col], ones16)
        return carry

    lax.fori_loop(0, epw // 16, hbody, 0)

    @pl.when(s == 0)
    def _():
        pltpu.sync_copy(zeros_hbm.at[pl.ds(0, NB)], shared_deg)

    plsc.subcore_barrier()

    def ibody(i, carry):
        idx80_v[pl.ds(i * 16, 16)] = lax.iota(jnp.int32, 16) + i * 16
        return carry

    lax.fori_loop(0, NB // 16, ibody, 0)
    pltpu.sync_copy(hist_v, shared_deg.at[idx80_v], add=True)
    plsc.subcore_barrier()

    @pl.when(s == 0)
    def _():
        pltpu.sync_copy(shared_deg, bounce_v)
        pltpu.sync_copy(bounce_v, out_hbm.at[c])


def _deg_kernel(dst_flat, zeros128):
    epw = dst_flat.shape[0] // NW
    f = functools.partial(
        pl.kernel,
        out_type=jax.ShapeDtypeStruct((NC, NB, 128), jnp.float32),
        mesh=_mesh,
        scratch_types=[
            pltpu.VMEM((epw,), jnp.int32),
            pltpu.VMEM((NB, 128), jnp.float32),
            pltpu.VMEM((NB,), jnp.int32),
            pltpu.VMEM((NB, 128), jnp.float32),
            pltpu.VMEM_SHARED((NB, 128), jnp.float32),
        ],
        compiler_params=pltpu.CompilerParams(needs_layout_passes=False),
    )(_deg_body)
    return f(dst_flat, zeros128)


def _spmm_body(h_hbm, src_hbm, dst_hbm, zeros_hbm, out_hbm, src_v, dst_v,
               buf, sem, acc):
    c = lax.axis_index("c")
    s = lax.axis_index("s")
    wid = s * NC + c
    cpw = src_hbm.shape[0] // NW
    rows_per_tile = NP // NS
    pltpu.sync_copy(src_hbm.at[pl.ds(wid * cpw, cpw)], src_v)
    pltpu.sync_copy(dst_hbm.at[pl.ds(wid * cpw, cpw)], dst_v)

    def zbody(i, carry):
        pltpu.sync_copy(zeros_hbm, acc.at[pl.ds(s * rows_per_tile + i * 128, 128)])
        return carry

    lax.fori_loop(0, rows_per_tile // 128, zbody, 0)
    plsc.subcore_barrier()

    def body(j, carry):
        pltpu.sync_copy(h_hbm.at[src_v.at[j]], buf)
        pltpu.sync_copy(buf, acc.at[dst_v.at[j]], add=True)
        return carry

    lax.fori_loop(0, cpw, body, 0)
    plsc.subcore_barrier()

    def wbody(i, carry):
        base = s * rows_per_tile + i * 128
        pltpu.sync_copy(acc.at[pl.ds(base, 128)], buf)
        pltpu.sync_copy(buf, out_hbm.at[c, pl.ds(base, 128)])
        return carry

    lax.fori_loop(0, rows_per_tile // 128, wbody, 0)


def _spmm(h, src2d, dst2d, zeros128):
    cpw = src2d.shape[0] // NW
    f = functools.partial(
        pl.kernel,
        out_type=jax.ShapeDtypeStruct((NC, NP, 128), jnp.float32),
        mesh=_mesh,
        scratch_types=[
            pltpu.VMEM((cpw, 128), jnp.int32),
            pltpu.VMEM((cpw, 128), jnp.int32),
            pltpu.VMEM((128, 128), jnp.float32),
            pltpu.SemaphoreType.DMA,
            pltpu.VMEM_SHARED((NP, 128), jnp.float32),
        ],
    )(_spmm_body)
    return f(h, src2d, dst2d, zeros128)


def _rsqrt_body(d_ref, o_ref):
    dtot = d_ref[0] + d_ref[1] + 1.0
    r = lax.rsqrt(dtot)
    flat = (lax.broadcasted_iota(jnp.int32, (NB, 128), 0) * 128
            + lax.broadcasted_iota(jnp.int32, (NB, 128), 1))
    o_ref[...] = jnp.where(flat < N, r, 0.0)


def _first_body(z_ref, w_ref, dv_ref, o_ref):
    o_ref[...] = dv_ref[...] * jnp.dot(z_ref[...], w_ref[...],
                                       preferred_element_type=jnp.float32)


def _mid_body(acc_ref, hp_ref, dv_ref, b_ref, g_ref, be_ref, wn_ref, o_ref):
    conv = dv_ref[...] * (acc_ref[0] + acc_ref[1] + hp_ref[...]) + b_ref[...]
    mask = lax.broadcasted_iota(jnp.int32, (NP, 1), 0) < N
    cm = jnp.where(mask, conv, 0.0)
    mean = jnp.sum(cm, axis=0, keepdims=True) * (1.0 / N)
    dlt = conv - mean
    var = jnp.sum(jnp.where(mask, dlt * dlt, 0.0), axis=0, keepdims=True) * (1.0 / N)
    y = dlt * lax.rsqrt(var + 1e-5) * g_ref[...] + be_ref[...]
    x = jnp.where(mask, jnp.maximum(y, 0.0), 0.0)
    o_ref[...] = dv_ref[...] * jnp.dot(x, wn_ref[...],
                                       preferred_element_type=jnp.float32)


def _final_body(acc_ref, hp_ref, dv_ref, b_ref, o_ref):
    o_ref[...] = (dv_ref[...] * (acc_ref[0] + acc_ref[1] + hp_ref[...])
                  + b_ref[...])


def _tc(body, out_shape, *args):
    return pl.pallas_call(body, out_shape=out_shape)(*args)


def kernel(z, W1, b1, g1, be1, W2, b2, g2, be2, W3, b3, edge_index):
    E = edge_index.shape[1]
    epad = NW * B * 8 * -(-E // (NW * B * 8))
    src = edge_index[0].astype(jnp.int32)
    dst = edge_index[1].astype(jnp.int32)
    fill = jnp.full((epad - E,), PAD_ROW, jnp.int32)
    srcp = jnp.concatenate([src, fill])
    dstp = jnp.concatenate([dst, fill])
    src2d = srcp.reshape(epad // 128, 128)
    dst2d = dstp.reshape(epad // 128, 128)
    zeros128 = jnp.zeros((128, 128), jnp.float32)
    z_pad = jnp.pad(z, ((0, NP - N), (0, 0)))

    deg2 = _deg_kernel(dstp, zeros128)
    dinv80 = _tc(_rsqrt_body, jax.ShapeDtypeStruct((NB, 128), jnp.float32), deg2)
    dv = dinv80.reshape(NP)[:, None]

    f32 = jnp.float32
    h1 = _tc(_first_body, jax.ShapeDtypeStruct((NP, D), f32), z_pad, W1, dv)
    a1 = _spmm(h1, src2d, dst2d, zeros128)
    h2 = _tc(_mid_body, jax.ShapeDtypeStruct((NP, D), f32),
             a1, h1, dv, b1[None], g1[None], be1[None], W2)
    a2 = _spmm(h2, src2d, dst2d, zeros128)
    h3 = _tc(_mid_body, jax.ShapeDtypeStruct((NP, D), f32),
             a2, h2, dv, b2[None], g2[None], be2[None], W3)
    a3 = _spmm(h3, src2d, dst2d, zeros128)
    out = _tc(_final_body, jax.ShapeDtypeStruct((NP, D), f32),
              a3, h3, dv, b3[None])
    return out[:N]
